# Initial kernel scaffold; baseline (speedup 1.0000x reference)
import jax
import jax.numpy as jnp
from jax.experimental import pallas as pl


def kernel(session_idx, subject_idx, session_table, subject_table, session_flag, subject_flag):
    raise NotImplementedError("write your pallas kernel here")


# speedup vs baseline: 1.2556x; 1.2556x over previous
import functools

import jax
import jax.numpy as jnp
from jax import lax
from jax.experimental import pallas as pl
from jax.experimental.pallas import tpu as pltpu
from jax.experimental.pallas import tpu_sc as plsc

BATCH = 4096
VOCAB = 1000
DIM = 128
LANES = 16

_info = plsc.get_sparse_core_info()
_NC, _NS = _info.num_cores, _info.num_subcores
_NW = _NC * _NS
_B_PER_W = BATCH // _NW

_mesh = plsc.VectorSubcoreMesh(core_axis_name="c", subcore_axis_name="s")


@functools.partial(
    pl.kernel,
    mesh=_mesh,
    out_type=jax.ShapeDtypeStruct((BATCH, 2, DIM), jnp.float32),
    scratch_types=[
        pltpu.VMEM((_B_PER_W,), jnp.int32),
        pltpu.VMEM((_B_PER_W,), jnp.int32),
        pltpu.VMEM((_B_PER_W, DIM), jnp.float32),
        pltpu.VMEM((_B_PER_W, DIM), jnp.float32),
        pltpu.VMEM((DIM,), jnp.float32),
        pltpu.VMEM((DIM,), jnp.float32),
        pltpu.VMEM((_B_PER_W, 2, DIM), jnp.float32),
        pltpu.SemaphoreType.DMA,
        pltpu.SemaphoreType.DMA,
    ],
)
def _ctx_emb_kernel(sess_idx_hbm, subj_idx_hbm, sess_tab_hbm, subj_tab_hbm,
                    sess_flag_hbm, subj_flag_hbm, out_hbm,
                    idx_s, idx_b, rows_s, rows_b, flag_s, flag_b,
                    stacked, sem_s, sem_b):
    wid = lax.axis_index("s") * _NC + lax.axis_index("c")
    base = wid * _B_PER_W

    pltpu.sync_copy(sess_idx_hbm.at[pl.ds(base, _B_PER_W)], idx_s)
    pltpu.sync_copy(subj_idx_hbm.at[pl.ds(base, _B_PER_W)], idx_b)
    cp_s = pltpu.async_copy(sess_tab_hbm.at[idx_s], rows_s, sem_s)
    cp_b = pltpu.async_copy(subj_tab_hbm.at[idx_b], rows_b, sem_b)
    pltpu.sync_copy(sess_flag_hbm, flag_s)
    pltpu.sync_copy(subj_flag_hbm, flag_b)
    cp_s.wait()
    cp_b.wait()

    fs = [flag_s[pl.ds(c * LANES, LANES)] for c in range(DIM // LANES)]
    fb = [flag_b[pl.ds(c * LANES, LANES)] for c in range(DIM // LANES)]

    def body(i, carry):
        for c in range(DIM // LANES):
            d = pl.ds(c * LANES, LANES)
            stacked[i, 0, d] = rows_s[i, d] + fs[c]
            stacked[i, 1, d] = rows_b[i, d] + fb[c]
        return carry

    lax.fori_loop(0, _B_PER_W, body, 0)

    pltpu.sync_copy(stacked, out_hbm.at[pl.ds(base, _B_PER_W)])


def kernel(session_idx, subject_idx, session_table, subject_table,
           session_flag, subject_flag):
    return _ctx_emb_kernel(session_idx, subject_idx, session_table,
                           subject_table, session_flag, subject_flag)


# speedup vs baseline: 1.4874x; 1.1846x over previous
import functools

import jax
import jax.numpy as jnp
from jax import lax
from jax.experimental import pallas as pl
from jax.experimental.pallas import tpu as pltpu
from jax.experimental.pallas import tpu_sc as plsc

BATCH = 4096
VOCAB = 1000
DIM = 128
LANES = 16

_info = plsc.get_sparse_core_info()
_NC, _NS = _info.num_cores, _info.num_subcores
_NW = _NC * _NS
_B_PER_W = BATCH // _NW
_NB = 4
_ROWS = _B_PER_W // _NB

_mesh = plsc.VectorSubcoreMesh(core_axis_name="c", subcore_axis_name="s")


@functools.partial(
    pl.kernel,
    mesh=_mesh,
    out_type=jax.ShapeDtypeStruct((BATCH, 2, DIM), jnp.float32),
    scratch_types=(
        [
            pltpu.VMEM((_B_PER_W,), jnp.int32),
            pltpu.VMEM((_B_PER_W,), jnp.int32),
            pltpu.VMEM((_B_PER_W, DIM), jnp.float32),
            pltpu.VMEM((_B_PER_W, DIM), jnp.float32),
            pltpu.VMEM((DIM,), jnp.float32),
            pltpu.VMEM((DIM,), jnp.float32),
            pltpu.VMEM((_B_PER_W, 2, DIM), jnp.float32),
        ]
        + [pltpu.SemaphoreType.DMA] * (3 * _NB)
    ),
)
def _ctx_emb_kernel(sess_idx_hbm, subj_idx_hbm, sess_tab_hbm, subj_tab_hbm,
                    sess_flag_hbm, subj_flag_hbm, out_hbm,
                    idx_s, idx_b, rows_s, rows_b, flag_s, flag_b,
                    stacked, *sems):
    sem_s = sems[0:_NB]
    sem_b = sems[_NB:2 * _NB]
    sem_o = sems[2 * _NB:3 * _NB]

    wid = lax.axis_index("s") * _NC + lax.axis_index("c")
    base = wid * _B_PER_W

    pltpu.sync_copy(sess_idx_hbm.at[pl.ds(base, _B_PER_W)], idx_s)
    pltpu.sync_copy(subj_idx_hbm.at[pl.ds(base, _B_PER_W)], idx_b)

    cp_s = []
    cp_b = []
    for k in range(_NB):
        r = pl.ds(k * _ROWS, _ROWS)
        cp_s.append(pltpu.async_copy(
            sess_tab_hbm.at[idx_s.at[r]], rows_s.at[r], sem_s[k]))
        cp_b.append(pltpu.async_copy(
            subj_tab_hbm.at[idx_b.at[r]], rows_b.at[r], sem_b[k]))

    pltpu.sync_copy(sess_flag_hbm, flag_s)
    pltpu.sync_copy(subj_flag_hbm, flag_b)
    fs = [flag_s[pl.ds(c * LANES, LANES)] for c in range(DIM // LANES)]
    fb = [flag_b[pl.ds(c * LANES, LANES)] for c in range(DIM // LANES)]

    cp_o = []
    for k in range(_NB):
        cp_s[k].wait()
        cp_b[k].wait()

        @plsc.parallel_loop(k * _ROWS, (k + 1) * _ROWS, unroll=2)
        def _body(i):
            for c in range(DIM // LANES):
                d = pl.ds(c * LANES, LANES)
                stacked[i, 0, d] = rows_s[i, d] + fs[c]
                stacked[i, 1, d] = rows_b[i, d] + fb[c]

        r = pl.ds(k * _ROWS, _ROWS)
        cp_o.append(pltpu.async_copy(
            stacked.at[r], out_hbm.at[pl.ds(base + k * _ROWS, _ROWS)],
            sem_o[k]))

    for k in range(_NB):
        cp_o[k].wait()


def kernel(session_idx, subject_idx, session_table, subject_table,
           session_flag, subject_flag):
    return _ctx_emb_kernel(session_idx, subject_idx, session_table,
                           subject_table, session_flag, subject_flag)
